# Initial kernel scaffold; baseline (speedup 1.0000x reference)
import jax
import jax.numpy as jnp
from jax.experimental import pallas as pl


def kernel(x, edge_index, node_to_subgraph, subgraph_to_graph, W1, b1, W2, b2, W3, b3, lin1_W, lin1_b, lin2_W, lin2_b):
    raise NotImplementedError("write your pallas kernel here")


# speedup vs baseline: 70.1614x; 70.1614x over previous
import functools

import jax
import jax.numpy as jnp
from jax import lax
from jax.experimental import pallas as pl
from jax.experimental.pallas import tpu as pltpu
from jax.experimental.pallas import tpu_sc as plsc

N = 10000
E = 320000
HIDDEN = 128
NUM_SUBGRAPHS = 1000
NUM_GRAPHS = 64
OUTPUT_DIM = 8

NC = 2
NS = 16
NPAD = 10240
RANGE = N // NC
LPAD = 5120
EC = E // NS
COLS = 320
TAIL = RANGE - (NS - 1) * COLS

_mesh = plsc.VectorSubcoreMesh(
    core_axis_name="c", subcore_axis_name="s", num_cores=NC, num_subcores=NS)

_f32 = jnp.float32
_i32 = jnp.int32


def _zero_ref(ref, nvec):
    z = jnp.zeros((16,), _f32)

    def body(i, _):
        ref[pl.ds(pl.multiple_of(i * 16, 16), 16)] = z

    lax.fori_loop(0, nvec, body, None)


def _newton_rsqrt(d):
    i = plsc.bitcast(d, _i32)
    i = 0x5F3759DF - lax.shift_right_logical(i, 1)
    y = plsc.bitcast(i, _f32)
    for _ in range(3):
        y = y * (1.5 - 0.5 * d * y * y)
    return y


@functools.partial(
    pl.kernel,
    mesh=_mesh,
    compiler_params=pltpu.CompilerParams(needs_layout_passes=False),
    out_type=(
        jax.ShapeDtypeStruct((NPAD,), _f32),
        jax.ShapeDtypeStruct((NPAD,), _f32),
    ),
    scratch_types=[
        pltpu.VMEM((EC,), _i32),
        pltpu.VMEM((LPAD,), _f32),
        pltpu.VMEM((COLS,), _f32),
        pltpu.VMEM((NS * COLS,), _f32),
        pltpu.VMEM((COLS,), _f32),
        pltpu.VMEM((COLS,), _f32),
        pltpu.VMEM_SHARED((NC * NS * LPAD,), _f32),
    ],
)
def _kernel_a(dst_hbm, x_hbm, dinv_hbm, sa0_hbm,
              dst_v, acc_v, xcol_v, red_v, dcol_v, scol_v, shared):
    c = lax.axis_index("c")
    s = lax.axis_index("s")
    base_c = c * RANGE
    col0 = s * COLS

    _zero_ref(acc_v, LPAD // 16)
    pltpu.sync_copy(dst_hbm.at[pl.ds(s * EC, EC)], dst_v)

    ones = jnp.ones((16,), _f32)

    def ebody(i, _):
        off = pl.multiple_of(i * 16, 16)
        d16 = dst_v[pl.ds(off, 16)]
        lidx = d16 - base_c
        m = (lidx >= 0) & (lidx < RANGE)
        plsc.addupdate_scatter(acc_v, [lidx], ones, mask=m)

    lax.fori_loop(0, EC // 16, ebody, None)

    pltpu.sync_copy(acc_v, shared.at[pl.ds((c * NS + s) * LPAD, LPAD)])
    plsc.subcore_barrier()

    for r in range(NS):
        pltpu.sync_copy(shared.at[pl.ds((c * NS + r) * LPAD + col0, COLS)],
                        red_v.at[pl.ds(r * COLS, COLS)])
    pltpu.sync_copy(x_hbm.at[pl.ds(base_c + col0, COLS)], xcol_v)

    def rbody(j, _):
        off = pl.multiple_of(j * 16, 16)
        tot = jnp.zeros((16,), _f32)
        for r in range(NS):
            tot = tot + red_v[pl.ds(pl.multiple_of(r * COLS + off, 16), 16)]
        d = tot + 1.0
        y = _newton_rsqrt(d)
        dcol_v[pl.ds(off, 16)] = y
        scol_v[pl.ds(off, 16)] = y * xcol_v[pl.ds(off, 16)]

    lax.fori_loop(0, COLS // 16, rbody, None)

    @pl.when(s < NS - 1)
    def _():
        pltpu.sync_copy(dcol_v, dinv_hbm.at[pl.ds(base_c + col0, COLS)])
        pltpu.sync_copy(scol_v, sa0_hbm.at[pl.ds(base_c + col0, COLS)])

    @pl.when(s == NS - 1)
    def _():
        pltpu.sync_copy(dcol_v.at[pl.ds(0, TAIL)],
                        dinv_hbm.at[pl.ds(base_c + col0, TAIL)])
        pltpu.sync_copy(scol_v.at[pl.ds(0, TAIL)],
                        sa0_hbm.at[pl.ds(base_c + col0, TAIL)])


@functools.partial(
    pl.kernel,
    mesh=_mesh,
    compiler_params=pltpu.CompilerParams(needs_layout_passes=False),
    out_type=(
        jax.ShapeDtypeStruct((NPAD,), _f32),
        jax.ShapeDtypeStruct((NPAD,), _f32),
    ),
    scratch_types=[
        pltpu.VMEM((EC,), _i32),
        pltpu.VMEM((EC,), _i32),
        pltpu.VMEM((NPAD,), _f32),
        pltpu.VMEM((NPAD,), _f32),
        pltpu.VMEM((LPAD,), _f32),
        pltpu.VMEM((LPAD,), _f32),
        pltpu.VMEM((NS * COLS,), _f32),
        pltpu.VMEM((COLS,), _f32),
        pltpu.VMEM((COLS,), _f32),
        pltpu.VMEM((COLS,), _f32),
        pltpu.VMEM_SHARED((NC * NS * LPAD,), _f32),
        pltpu.VMEM_SHARED((NC * NS * LPAD,), _f32),
    ],
)
def _kernel_b(src_hbm, dst_hbm, dinv_hbm, sa_hbm, sb_hbm, oa_hbm, ob_hbm,
              src_v, dst_v, sav, sbv, acca, accb, red_v, dcol_v,
              oca_v, ocb_v, shared_a, shared_b):
    c = lax.axis_index("c")
    s = lax.axis_index("s")
    base_c = c * RANGE
    col0 = s * COLS

    _zero_ref(acca, LPAD // 16)
    _zero_ref(accb, LPAD // 16)
    pltpu.sync_copy(src_hbm.at[pl.ds(s * EC, EC)], src_v)
    pltpu.sync_copy(dst_hbm.at[pl.ds(s * EC, EC)], dst_v)
    pltpu.sync_copy(sa_hbm, sav)
    pltpu.sync_copy(sb_hbm, sbv)

    def ebody(i, _):
        off = pl.multiple_of(i * 16, 16)
        s16 = src_v[pl.ds(off, 16)]
        d16 = dst_v[pl.ds(off, 16)]
        va = plsc.load_gather(sav, [s16])
        vb = plsc.load_gather(sbv, [s16])
        lidx = d16 - base_c
        m = (lidx >= 0) & (lidx < RANGE)
        plsc.addupdate_scatter(acca, [lidx], va, mask=m)
        plsc.addupdate_scatter(accb, [lidx], vb, mask=m)

    lax.fori_loop(0, EC // 16, ebody, None)

    pltpu.sync_copy(acca, shared_a.at[pl.ds((c * NS + s) * LPAD, LPAD)])
    pltpu.sync_copy(accb, shared_b.at[pl.ds((c * NS + s) * LPAD, LPAD)])
    plsc.subcore_barrier()

    pltpu.sync_copy(dinv_hbm.at[pl.ds(base_c + col0, COLS)], dcol_v)

    def _reduce(shared, ocol_v, scell):
        for r in range(NS):
            pltpu.sync_copy(
                shared.at[pl.ds((c * NS + r) * LPAD + col0, COLS)],
                red_v.at[pl.ds(r * COLS, COLS)])

        def rbody(j, _):
            off = pl.multiple_of(j * 16, 16)
            tot = jnp.zeros((16,), _f32)
            for r in range(NS):
                tot = tot + red_v[
                    pl.ds(pl.multiple_of(r * COLS + off, 16), 16)]
            dv = dcol_v[pl.ds(off, 16)]
            goff = base_c + col0 + j * 16
            tot = tot + plsc.load_gather(
                scell, [goff + lax.iota(_i32, 16)])
            ocol_v[pl.ds(off, 16)] = dv * dv * tot

        lax.fori_loop(0, COLS // 16, rbody, None)

    _reduce(shared_a, oca_v, sav)
    _reduce(shared_b, ocb_v, sbv)

    @pl.when(s < NS - 1)
    def _():
        pltpu.sync_copy(oca_v, oa_hbm.at[pl.ds(base_c + col0, COLS)])
        pltpu.sync_copy(ocb_v, ob_hbm.at[pl.ds(base_c + col0, COLS)])

    @pl.when(s == NS - 1)
    def _():
        pltpu.sync_copy(oca_v.at[pl.ds(0, TAIL)],
                        oa_hbm.at[pl.ds(base_c + col0, TAIL)])
        pltpu.sync_copy(ocb_v.at[pl.ds(0, TAIL)],
                        ob_hbm.at[pl.ds(base_c + col0, TAIL)])


_NPW = NPAD // (NC * NS)

@functools.partial(
    pl.kernel,
    mesh=_mesh,
    compiler_params=pltpu.CompilerParams(needs_layout_passes=False),
    out_type=jax.ShapeDtypeStruct((NC * NS * 4 * NUM_GRAPHS,), _f32),
    scratch_types=[
        pltpu.VMEM((_NPW,), _i32),
        pltpu.VMEM((NUM_SUBGRAPHS,), _i32),
        pltpu.VMEM((_NPW,), _f32),
        pltpu.VMEM((_NPW,), _f32),
        pltpu.VMEM((_NPW,), _f32),
        pltpu.VMEM((_NPW,), _f32),
        pltpu.VMEM((16 * 4 * NUM_GRAPHS,), _f32),
        pltpu.VMEM((4 * NUM_GRAPHS,), _f32),
    ],
)
def _kernel_c(n2s_hbm, s2g_hbm, dinv_hbm, sa3_hbm, sb2_hbm, sb1_hbm,
              part_hbm, n2s_v, s2g_v, v3, v2, v1, dv, acc, out_v):
    c = lax.axis_index("c")
    s = lax.axis_index("s")
    wid = s * NC + c
    nbase = wid * _NPW

    pltpu.sync_copy(n2s_hbm.at[pl.ds(nbase, _NPW)], n2s_v)
    pltpu.sync_copy(s2g_hbm, s2g_v)
    pltpu.sync_copy(sa3_hbm.at[pl.ds(nbase, _NPW)], v3)
    pltpu.sync_copy(sb2_hbm.at[pl.ds(nbase, _NPW)], v2)
    pltpu.sync_copy(sb1_hbm.at[pl.ds(nbase, _NPW)], v1)
    pltpu.sync_copy(dinv_hbm.at[pl.ds(nbase, _NPW)], dv)

    z = jnp.zeros((16,), _f32)
    _Q = 4 * NUM_GRAPHS
    for k in range(16 * _Q // 16):
        acc[pl.ds(k * 16, 16)] = z

    lane = lax.iota(_i32, 16)
    lbase = lane * _Q
    ones = jnp.ones((16,), _f32)

    def nbody(i, _):
        off = pl.multiple_of(i * 16, 16)
        sg = jnp.clip(n2s_v[pl.ds(off, 16)], 0, NUM_SUBGRAPHS - 1)
        g16 = plsc.load_gather(s2g_v, [sg]) + lbase
        gid = nbase + i * 16 + lane
        m = gid < N
        rdv = 1.0 / dv[pl.ds(off, 16)]
        plsc.addupdate_scatter(acc, [g16], v3[pl.ds(off, 16)] * rdv, mask=m)
        plsc.addupdate_scatter(acc, [g16 + NUM_GRAPHS],
                               v2[pl.ds(off, 16)] * rdv, mask=m)
        plsc.addupdate_scatter(acc, [g16 + 2 * NUM_GRAPHS],
                               v1[pl.ds(off, 16)] * rdv, mask=m)
        plsc.addupdate_scatter(acc, [g16 + 3 * NUM_GRAPHS], ones, mask=m)

    lax.fori_loop(0, _NPW // 16, nbody, None)

    for j in range(_Q // 16):
        tot = jnp.zeros((16,), _f32)
        for r in range(16):
            tot = tot + acc[pl.ds(r * _Q + j * 16, 16)]
        out_v[pl.ds(j * 16, 16)] = tot

    pltpu.sync_copy(out_v, part_hbm.at[pl.ds(wid * _Q, _Q)])


def _kernel_d_body(part_ref, w1_ref, b1_ref, w2_ref, b2_ref, w3_ref, b3_ref,
                   l1w_ref, l1b_ref, l2w_ref, l2b_ref, out_ref):
    pooled = jnp.sum(part_ref[...], axis=0)
    u = jnp.concatenate([w1_ref[...], b1_ref[...]], axis=0)
    v = jnp.dot(u, w2_ref[...], preferred_element_type=_f32, precision=lax.Precision.HIGHEST)
    r = jnp.concatenate([v, b2_ref[...]], axis=0)
    rw = jnp.dot(r, w3_ref[...], preferred_element_type=_f32, precision=lax.Precision.HIGHEST)
    cm = jnp.concatenate([rw, b3_ref[...]], axis=0)
    g = lax.dot_general(pooled, cm, (((0,), (0,)), ((), ())),
                        preferred_element_type=_f32,
                        precision=lax.Precision.HIGHEST)
    h = jnp.maximum(
        jnp.dot(g, l1w_ref[...], preferred_element_type=_f32, precision=lax.Precision.HIGHEST) + l1b_ref[...],
        0.0)
    o = jnp.dot(h, l2w_ref[...], preferred_element_type=_f32, precision=lax.Precision.HIGHEST) + l2b_ref[...]
    mx = jnp.max(o, axis=1, keepdims=True)
    ex = jnp.exp(o - mx)
    lse = jnp.log(jnp.sum(ex, axis=1, keepdims=True))
    out_ref[...] = o - mx - lse


_kernel_d = pl.pallas_call(
    _kernel_d_body,
    out_shape=jax.ShapeDtypeStruct((NUM_GRAPHS, OUTPUT_DIM), _f32),
)


def kernel(x, edge_index, node_to_subgraph, subgraph_to_graph,
           W1, b1, W2, b2, W3, b3, lin1_W, lin1_b, lin2_W, lin2_b):
    src = edge_index[0].astype(_i32)
    dst = edge_index[1].astype(_i32)
    x0 = jnp.concatenate(
        [x[:, 0].astype(_f32), jnp.zeros((NPAD - N,), _f32)])
    n2s = jnp.concatenate(
        [node_to_subgraph.astype(_i32), jnp.zeros((NPAD - N,), _i32)])
    s2g = subgraph_to_graph.astype(_i32)

    dinv, sa0 = _kernel_a(dst, x0)
    sa1, sb1 = _kernel_b(src, dst, dinv, sa0, dinv)
    sa2, sb2 = _kernel_b(src, dst, dinv, sa1, sb1)
    sa3, _ = _kernel_b(src, dst, dinv, sa2, sb2)
    parts = _kernel_c(n2s, s2g, dinv, sa3, sb2, sb1)
    parts = parts.reshape(NC * NS, 4, NUM_GRAPHS)

    return _kernel_d(
        parts, W1, b1.reshape(1, HIDDEN), W2, b2.reshape(1, HIDDEN),
        W3, b3.reshape(1, HIDDEN), lin1_W, lin1_b.reshape(1, HIDDEN),
        lin2_W, lin2_b.reshape(1, OUTPUT_DIM))
